# initial kernel scaffold (unmeasured)
import jax
import jax.numpy as jnp
from jax import lax
from jax.experimental import pallas as pl
from jax.experimental.pallas import tpu as pltpu

N_DEV = 16
N_RIGHT = 8
N_LEFT = N_DEV - 1 - N_RIGHT


def kernel(x, w_mat, scale_x, scale_w):
    m_per, k = x.shape
    _, n_per = w_mat.shape
    m_total = m_per * N_DEV

    def body(x_ref, w_ref, sx_ref, sw_ref, out_ref,
             comm_r, comm_l, send_r, recv_r, send_l, recv_l):
        my = lax.axis_index("i")
        right = lax.rem(my + 1, N_DEV)
        left = lax.rem(my + N_DEV - 1, N_DEV)

        barrier = pltpu.get_barrier_semaphore()
        pl.semaphore_signal(barrier, inc=1, device_id=(left,),
                            device_id_type=pl.DeviceIdType.MESH)
        pl.semaphore_signal(barrier, inc=1, device_id=(right,),
                            device_id_type=pl.DeviceIdType.MESH)
        pl.semaphore_wait(barrier, 2)

        scale = sx_ref[0] * sw_ref[0]

        def matmul_store(chunk, origin):
            acc = lax.dot_general(
                chunk, w_ref[...], (((1,), (0,)), ((), ())),
                preferred_element_type=jnp.float32,
            )
            out_ref[pl.ds(origin * m_per, m_per), :] = jnp.maximum(
                acc * scale, 0.0)

        def make_rdma(src, dst, ssem, rsem, tgt):
            return pltpu.make_async_remote_copy(
                src_ref=src, dst_ref=dst, send_sem=ssem, recv_sem=rsem,
                device_id=(tgt,), device_id_type=pl.DeviceIdType.MESH,
            )

        rdmas_r = [
            make_rdma(x_ref if h == 0 else comm_r.at[h - 1],
                      comm_r.at[h], send_r.at[h], recv_r.at[h], right)
            for h in range(N_RIGHT)
        ]
        rdmas_l = [
            make_rdma(x_ref if h == 0 else comm_l.at[h - 1],
                      comm_l.at[h], send_l.at[h], recv_l.at[h], left)
            for h in range(N_LEFT)
        ]

        rdmas_r[0].start()
        rdmas_l[0].start()

        matmul_store(x_ref[...], my)

        for h in range(N_RIGHT):
            rdmas_r[h].wait()
            if h + 1 < N_RIGHT:
                rdmas_r[h + 1].start()
            if h < N_LEFT:
                rdmas_l[h].wait()
                if h + 1 < N_LEFT:
                    rdmas_l[h + 1].start()
            matmul_store(comm_r[h], lax.rem(my + N_DEV - 1 - h, N_DEV))
            if h < N_LEFT:
                matmul_store(comm_l[h], lax.rem(my + 1 + h, N_DEV))

    return pl.pallas_call(
        body,
        out_shape=jax.ShapeDtypeStruct((m_total, n_per), jnp.float32),
        in_specs=[
            pl.BlockSpec(memory_space=pltpu.VMEM),
            pl.BlockSpec(memory_space=pltpu.VMEM),
            pl.BlockSpec(memory_space=pltpu.SMEM),
            pl.BlockSpec(memory_space=pltpu.SMEM),
        ],
        out_specs=pl.BlockSpec(memory_space=pltpu.VMEM),
        scratch_shapes=[
            pltpu.VMEM((N_RIGHT, m_per, k), x.dtype),
            pltpu.VMEM((N_LEFT, m_per, k), x.dtype),
            pltpu.SemaphoreType.DMA((N_RIGHT,)),
            pltpu.SemaphoreType.DMA((N_RIGHT,)),
            pltpu.SemaphoreType.DMA((N_LEFT,)),
            pltpu.SemaphoreType.DMA((N_LEFT,)),
        ],
        compiler_params=pltpu.CompilerParams(collective_id=0),
    )(x, w_mat, scale_x, scale_w)


# baseline (device time: 123970 ns/iter reference)
import jax
import jax.numpy as jnp
from jax import lax
from jax.experimental import pallas as pl
from jax.experimental.pallas import tpu as pltpu

N_DEV = 16
N_RIGHT = 8
N_LEFT = N_DEV - 1 - N_RIGHT


def kernel(x, w_mat, scale_x, scale_w):
    m_per, k = x.shape
    _, n_per = w_mat.shape
    m_total = m_per * N_DEV

    fp8 = jnp.float8_e5m2

    def body(x_ref, w_ref, sx_ref, sw_ref, out_ref,
             x8, w8, comm_r, comm_l, send_r, recv_r, send_l, recv_l):
        my = lax.axis_index("i")
        right = lax.rem(my + 1, N_DEV)
        left = lax.rem(my + N_DEV - 1, N_DEV)

        barrier = pltpu.get_barrier_semaphore()
        pl.semaphore_signal(barrier, inc=1, device_id=(left,),
                            device_id_type=pl.DeviceIdType.MESH)
        pl.semaphore_signal(barrier, inc=1, device_id=(right,),
                            device_id_type=pl.DeviceIdType.MESH)

        x8[...] = x_ref[...].astype(fp8)
        w8[...] = w_ref[...].astype(fp8)

        pl.semaphore_wait(barrier, 2)

        scale = sx_ref[0] * sw_ref[0]

        def matmul_store(chunk, origin):
            acc = lax.dot_general(
                chunk, w8[...], (((1,), (0,)), ((), ())),
                preferred_element_type=jnp.float32,
            )
            out_ref[pl.ds(origin * m_per, m_per), :] = jnp.maximum(
                acc * scale, 0.0)

        def make_rdma(src, dst, ssem, rsem, tgt):
            return pltpu.make_async_remote_copy(
                src_ref=src, dst_ref=dst, send_sem=ssem, recv_sem=rsem,
                device_id=(tgt,), device_id_type=pl.DeviceIdType.MESH,
            )

        rdmas_r = [
            make_rdma(x8 if h == 0 else comm_r.at[h - 1],
                      comm_r.at[h], send_r.at[h], recv_r.at[h], right)
            for h in range(N_RIGHT)
        ]
        rdmas_l = [
            make_rdma(x8 if h == 0 else comm_l.at[h - 1],
                      comm_l.at[h], send_l.at[h], recv_l.at[h], left)
            for h in range(N_LEFT)
        ]

        rdmas_r[0].start()
        rdmas_l[0].start()

        matmul_store(x8[...], my)

        for h in range(N_RIGHT):
            rdmas_r[h].wait()
            if h + 1 < N_RIGHT:
                rdmas_r[h + 1].start()
            if h < N_LEFT:
                rdmas_l[h].wait()
                if h + 1 < N_LEFT:
                    rdmas_l[h + 1].start()
            matmul_store(comm_r[h], lax.rem(my + N_DEV - 1 - h, N_DEV))
            if h < N_LEFT:
                matmul_store(comm_l[h], lax.rem(my + 1 + h, N_DEV))

    return pl.pallas_call(
        body,
        out_shape=jax.ShapeDtypeStruct((m_total, n_per), jnp.float32),
        in_specs=[
            pl.BlockSpec(memory_space=pltpu.VMEM),
            pl.BlockSpec(memory_space=pltpu.VMEM),
            pl.BlockSpec(memory_space=pltpu.SMEM),
            pl.BlockSpec(memory_space=pltpu.SMEM),
        ],
        out_specs=pl.BlockSpec(memory_space=pltpu.VMEM),
        scratch_shapes=[
            pltpu.VMEM((m_per, k), fp8),
            pltpu.VMEM((k, n_per), fp8),
            pltpu.VMEM((N_RIGHT, m_per, k), fp8),
            pltpu.VMEM((N_LEFT, m_per, k), fp8),
            pltpu.SemaphoreType.DMA((N_RIGHT,)),
            pltpu.SemaphoreType.DMA((N_RIGHT,)),
            pltpu.SemaphoreType.DMA((N_LEFT,)),
            pltpu.SemaphoreType.DMA((N_LEFT,)),
        ],
        compiler_params=pltpu.CompilerParams(
            collective_id=0,
            vmem_limit_bytes=100 * 1024 * 1024,
        ),
    )(x, w_mat, scale_x, scale_w)


# device time: 108521 ns/iter; 1.1424x vs baseline; 1.1424x over previous
import jax
import jax.numpy as jnp
from jax import lax
from jax.experimental import pallas as pl
from jax.experimental.pallas import tpu as pltpu

N_DEV = 16
H = N_DEV // 2
N_SEG = 2


def kernel(x, w_mat, scale_x, scale_w):
    m_per, k = x.shape
    _, n_per = w_mat.shape
    m_total = m_per * N_DEV
    m_seg = m_per // N_SEG

    fp8 = jnp.float8_e5m2

    r_set = {(h, j) for h in range(H) for j in range(N_SEG)
             if h < H - 1 or j == 0}
    l_set = {(h, j) for h in range(H) for j in range(N_SEG)
             if h < H - 1 or j == 1}

    def body(x_ref, w_ref, sx_ref, sw_ref, out_ref,
             x8, w8, comm_r, comm_l, send_r, recv_r, send_l, recv_l):
        my = lax.axis_index("i")
        right = lax.rem(my + 1, N_DEV)
        left = lax.rem(my + N_DEV - 1, N_DEV)

        barrier = pltpu.get_barrier_semaphore()
        pl.semaphore_signal(barrier, inc=1, device_id=(left,),
                            device_id_type=pl.DeviceIdType.MESH)
        pl.semaphore_signal(barrier, inc=1, device_id=(right,),
                            device_id_type=pl.DeviceIdType.MESH)

        x8[...] = x_ref[...].astype(fp8)

        pl.semaphore_wait(barrier, 2)

        def make_rdma(src, dst, ssem, rsem, tgt):
            return pltpu.make_async_remote_copy(
                src_ref=src, dst_ref=dst, send_sem=ssem, recv_sem=rsem,
                device_id=(tgt,), device_id_type=pl.DeviceIdType.MESH,
            )

        def rdma_r(h, j):
            src = x8.at[pl.ds(j * m_seg, m_seg)] if h == 0 \
                else comm_r.at[h - 1, j]
            return make_rdma(src, comm_r.at[h, j],
                             send_r.at[h, j], recv_r.at[h, j], right)

        def rdma_l(h, j):
            src = x8.at[pl.ds(j * m_seg, m_seg)] if h == 0 \
                else comm_l.at[h - 1, j]
            return make_rdma(src, comm_l.at[h, j],
                             send_l.at[h, j], recv_l.at[h, j], left)

        rdmas_r = {hj: rdma_r(*hj) for hj in r_set}
        rdmas_l = {hj: rdma_l(*hj) for hj in l_set}

        for j in range(N_SEG):
            rdmas_r[(0, j)].start()
            rdmas_l[(0, j)].start()

        w8[...] = w_ref[...].astype(fp8)
        scale = sx_ref[0] * sw_ref[0]

        def matmul_store(chunk, row_base):
            acc = lax.dot_general(
                chunk, w8[...], (((1,), (0,)), ((), ())),
                preferred_element_type=jnp.float32,
            )
            out_ref[pl.ds(row_base, m_seg), :] = jnp.maximum(
                acc * scale, 0.0)

        for j in range(N_SEG):
            matmul_store(x8[pl.ds(j * m_seg, m_seg)],
                         my * m_per + j * m_seg)

        for h in range(H):
            origin_r = lax.rem(my + N_DEV - 1 - h, N_DEV)
            origin_l = lax.rem(my + 1 + h, N_DEV)
            for j in range(N_SEG):
                if (h, j) in r_set:
                    rdmas_r[(h, j)].wait()
                    if (h + 1, j) in r_set:
                        rdmas_r[(h + 1, j)].start()
                if (h, j) in l_set:
                    rdmas_l[(h, j)].wait()
                    if (h + 1, j) in l_set:
                        rdmas_l[(h + 1, j)].start()
                if (h, j) in r_set:
                    matmul_store(comm_r[h, j],
                                 origin_r * m_per + j * m_seg)
                if (h, j) in l_set:
                    matmul_store(comm_l[h, j],
                                 origin_l * m_per + j * m_seg)

    return pl.pallas_call(
        body,
        out_shape=jax.ShapeDtypeStruct((m_total, n_per), jnp.float32),
        in_specs=[
            pl.BlockSpec(memory_space=pltpu.VMEM),
            pl.BlockSpec(memory_space=pltpu.VMEM),
            pl.BlockSpec(memory_space=pltpu.SMEM),
            pl.BlockSpec(memory_space=pltpu.SMEM),
        ],
        out_specs=pl.BlockSpec(memory_space=pltpu.VMEM),
        scratch_shapes=[
            pltpu.VMEM((m_per, k), fp8),
            pltpu.VMEM((k, n_per), fp8),
            pltpu.VMEM((H, N_SEG, m_seg, k), fp8),
            pltpu.VMEM((H, N_SEG, m_seg, k), fp8),
            pltpu.SemaphoreType.DMA((H, N_SEG)),
            pltpu.SemaphoreType.DMA((H, N_SEG)),
            pltpu.SemaphoreType.DMA((H, N_SEG)),
            pltpu.SemaphoreType.DMA((H, N_SEG)),
        ],
        compiler_params=pltpu.CompilerParams(
            collective_id=0,
            vmem_limit_bytes=100 * 1024 * 1024,
        ),
    )(x, w_mat, scale_x, scale_w)
